# baseline (speedup 1.0000x reference)
import functools

import jax
import jax.numpy as jnp
from jax import lax
from jax.experimental import pallas as pl
from jax.experimental.pallas import tpu as pltpu
from jax.experimental.pallas import tpu_sc as plsc

N = 10000
E = 320000
D = 128

NC = 2
NS = 16
L = 16

NP = 10240
W_RED = NP // NS

EPT = E // NS
CHUNK = 128
NFULL = EPT // CHUNK
REM = EPT - NFULL * CHUNK

GROUP = 26

DEG_CH = 800

_SC_PARAMS = pltpu.CompilerParams(needs_layout_passes=False)


def _build_degree_kernel():
  mesh = plsc.VectorSubcoreMesh(core_axis_name="c", subcore_axis_name="s")

  @functools.partial(
      pl.kernel,
      out_type=jax.ShapeDtypeStruct((4 * NP,), jnp.float32),
      mesh=mesh,
      compiler_params=_SC_PARAMS,
      scratch_types=[
          pltpu.VMEM((DEG_CH,), jnp.int32),
          pltpu.VMEM((DEG_CH,), jnp.int32),
          pltpu.VMEM((NP,), jnp.float32),
          pltpu.VMEM((NP,), jnp.float32),
          pltpu.VMEM_SHARED((NS * NP,), jnp.float32),
          pltpu.VMEM_SHARED((NS * NP,), jnp.float32),
          pltpu.VMEM((NS * W_RED,), jnp.float32),
          pltpu.VMEM((W_RED,), jnp.float32),
      ],
  )
  def deg_kernel(src_hbm, dst_hbm, out_hbm, src_v, dst_v, ho, hi,
                 parts_o, parts_i, redbuf, res):
    c = lax.axis_index("c")
    s = lax.axis_index("s")
    zeros16 = jnp.zeros((L,), jnp.float32)

    @pl.loop(0, NP // L)
    def _zero(i):
      ho[pl.ds(i * L, L)] = zeros16
      hi[pl.ds(i * L, L)] = zeros16

    base = c * E + s * EPT

    @pl.loop(0, EPT // DEG_CH)
    def _edges(i):
      pltpu.sync_copy(src_hbm.at[pl.ds(base + i * DEG_CH, DEG_CH)], src_v)
      pltpu.sync_copy(dst_hbm.at[pl.ds(base + i * DEG_CH, DEG_CH)], dst_v)

      @pl.loop(0, DEG_CH // L)
      def _vecs(j):
        sidx = src_v[pl.ds(j * L, L)]
        scnt, slast = plsc.scan_count(sidx)
        plsc.addupdate_scatter(ho, [sidx], scnt.astype(jnp.float32),
                               mask=slast)
        didx = dst_v[pl.ds(j * L, L)]
        dcnt, dlast = plsc.scan_count(didx)
        plsc.addupdate_scatter(hi, [didx], dcnt.astype(jnp.float32),
                               mask=dlast)

    pltpu.sync_copy(ho, parts_o.at[pl.ds(s * NP, NP)])
    pltpu.sync_copy(hi, parts_i.at[pl.ds(s * NP, NP)])
    plsc.subcore_barrier()

    for h, parts in ((0, parts_o), (1, parts_i)):
      for p in range(NS):
        pltpu.sync_copy(parts.at[pl.ds(p * NP + s * W_RED, W_RED)],
                        redbuf.at[pl.ds(p * W_RED, W_RED)])

      @pl.loop(0, W_RED // L)
      def _red(j):
        acc = redbuf[pl.ds(j * L, L)]
        for p in range(1, NS):
          acc = acc + redbuf[pl.ds(p * W_RED + j * L, L)]
        res[pl.ds(j * L, L)] = acc

      pltpu.sync_copy(res,
                      out_hbm.at[pl.ds((2 * c + h) * NP + s * W_RED, W_RED)])

  return deg_kernel


def _build_conv_kernel():
  mesh = plsc.VectorSubcoreMesh(core_axis_name="c", subcore_axis_name="s")

  @functools.partial(
      pl.kernel,
      out_type=jax.ShapeDtypeStruct((2 * NP, D), jnp.float32),
      mesh=mesh,
      compiler_params=_SC_PARAMS,
      scratch_types=[
          pltpu.VMEM((GROUP * CHUNK,), jnp.int32),
          pltpu.VMEM((GROUP * CHUNK,), jnp.int32),
          pltpu.VMEM((CHUNK,), jnp.int32),
          pltpu.VMEM((CHUNK,), jnp.int32),
          pltpu.VMEM((CHUNK, D), jnp.float32),
          pltpu.VMEM((CHUNK, D), jnp.float32),
          pltpu.VMEM((REM,), jnp.int32),
          pltpu.VMEM((REM,), jnp.int32),
          pltpu.VMEM((REM, D), jnp.float32),
          pltpu.VMEM_SHARED((NP, D), jnp.float32),
          pltpu.SemaphoreType.DMA,
          pltpu.SemaphoreType.DMA,
          pltpu.SemaphoreType.DMA,
          pltpu.SemaphoreType.DMA,
          pltpu.SemaphoreType.DMA,
      ],
  )
  def conv_kernel(hs_hbm, src_hbm, dst_hbm, out_hbm, sv_g, dv_g, dv0, dv1,
                  rows0, rows1, src_r, dst_r, rows_r, acc,
                  gsem0, gsem1, ssem0, ssem1, rsem):
    c = lax.axis_index("c")
    s = lax.axis_index("s")
    zeros16 = jnp.zeros((L,), jnp.float32)

    ebase = c * E + s * EPT
    roff = c * N

    @pl.loop(0, CHUNK)
    def _zfill(r):
      for jj in range(D // L):
        rows0[r, pl.ds(jj * L, L)] = zeros16

    row0 = s * W_RED
    for k in range(W_RED // CHUNK):
      pltpu.sync_copy(rows0, acc.at[pl.ds(row0 + k * CHUNK, CHUNK)])
    plsc.subcore_barrier()

    slots = ((dv0, rows0, gsem0, ssem0), (dv1, rows1, gsem1, ssem1))

    def prep_slot(q, slot, drain):
      dv, rows, gsem, ssem = slots[slot]
      if drain:
        pltpu.make_async_copy(rows, acc.at[dv], ssem).wait()
      for j in range(CHUNK // L):
        dv[pl.ds(j * L, L)] = dv_g[pl.ds(q * CHUNK + j * L, L)]
      return pltpu.async_copy(hs_hbm.at[sv_g.at[pl.ds(q * CHUNK, CHUNK)]],
                              rows, gsem)

    def do_pair(q, drain):
      g0 = prep_slot(q, 0, drain)
      g1 = prep_slot(q + 1, 1, drain)
      g0.wait()
      pltpu.async_copy(rows0, acc.at[dv0], ssem0, add=True)
      g1.wait()
      pltpu.async_copy(rows1, acc.at[dv1], ssem1, add=True)

    def load_group(g):
      b0 = ebase + g * GROUP * CHUNK
      pltpu.sync_copy(src_hbm.at[pl.ds(b0, GROUP * CHUNK)], sv_g)
      pltpu.sync_copy(dst_hbm.at[pl.ds(b0, GROUP * CHUNK)], dv_g)

      @pl.loop(0, GROUP * CHUNK // L)
      def _adj(i):
        sv_g[pl.ds(i * L, L)] = sv_g[pl.ds(i * L, L)] + roff

    load_group(0)
    do_pair(0, False)
    for q in range(2, GROUP, 2):
      do_pair(q, True)

    @pl.loop(1, NFULL // GROUP)
    def _groups(g):
      load_group(g)
      for q in range(0, GROUP, 2):
        do_pair(q, True)

    b0 = ebase + NFULL * CHUNK
    pltpu.sync_copy(src_hbm.at[pl.ds(b0, REM)], src_r)
    pltpu.sync_copy(dst_hbm.at[pl.ds(b0, REM)], dst_r)
    for j in range(REM // L):
      src_r[pl.ds(j * L, L)] = src_r[pl.ds(j * L, L)] + roff
    pltpu.async_copy(hs_hbm.at[src_r], rows_r, rsem).wait()
    pltpu.sync_copy(rows_r, acc.at[dst_r], add=True)

    for dv, rows, gsem, ssem in slots:
      pltpu.make_async_copy(rows, acc.at[dv], ssem).wait()

    plsc.subcore_barrier()
    pltpu.sync_copy(acc.at[pl.ds(row0, W_RED)],
                    out_hbm.at[pl.ds(c * NP + row0, W_RED)])

  return conv_kernel


def _ns_nd(deg_full):
  g = pl.program_id(0)
  rs = lax.rsqrt(jnp.maximum(deg_full[:, :N], 1.0))
  ns = jnp.where(g == 0, rs[0], rs[2])
  nd = jnp.where(g == 0, rs[1], rs[3])
  return ns, nd


def _prep_body(xs_ref, w1_ref, deg_ref, hs_ref):
  ns, _ = _ns_nd(deg_ref[...])
  h = jnp.dot(xs_ref[0], w1_ref[...], preferred_element_type=jnp.float32, precision=lax.Precision.HIGHEST)
  hs_ref[0] = h * ns[:, None]


def _mid_body(agg_ref, deg_ref, w2_ref, b1_ref, out_ref):
  ns, nd = _ns_nd(deg_ref[...])
  a = agg_ref[0, :N]
  h = jnp.maximum(a * nd[:, None] + b1_ref[...], 0.0)
  h2 = jnp.dot(h, w2_ref[...], preferred_element_type=jnp.float32, precision=lax.Precision.HIGHEST)
  out_ref[0] = h2 * ns[:, None]


def _sigmoid(x):
  return 1.0 / (1.0 + jnp.exp(-x))


def _tail_body(agg_ref, deg_ref, b2_ref, ro_ref, co_ref,
               cm_ref, rm_ref, cmk_ref, z1_ref, z2_ref):
  degs = deg_ref[...]
  nd1 = lax.rsqrt(jnp.maximum(degs[1, :N], 1.0))
  nd2 = lax.rsqrt(jnp.maximum(degs[3, :N], 1.0))
  b2 = b2_ref[...]
  z1 = agg_ref[0, :N] * nd1[:, None] + b2
  z2 = agg_ref[1, :N] * nd2[:, None] + b2
  z1_ref[...] = z1
  z2_ref[...] = z2
  n = jnp.float32(N)
  mu1 = jnp.sum(z1, axis=0) / n
  mu2 = jnp.sum(z2, axis=0) / n
  s1 = jnp.sum(z1 * z1, axis=0)
  s2 = jnp.sum(z2 * z2, axis=0)
  var1 = (s1 - n * mu1 * mu1) / (n - 1.0)
  var2 = (s2 - n * mu2 * mu2) / (n - 1.0)
  sd1 = jnp.sqrt(jnp.maximum(var1, 0.0)) + 1e-6
  sd2 = jnp.sqrt(jnp.maximum(var2, 0.0)) + 1e-6
  S = lax.dot_general(z1, z2, (((0,), (0,)), ((), ())),
                      preferred_element_type=jnp.float32,
                      precision=lax.Precision.HIGHEST)
  C = (S - n * mu1[:, None] * mu2[None, :]) / (n * sd1[:, None] * sd2[None, :])
  row_score = jnp.mean(jnp.abs(C), axis=1)
  col_score = jnp.mean(jnp.abs(C), axis=0)
  rm = _sigmoid(50.0 * (row_score + ro_ref[0] - 0.05))
  cm = _sigmoid(50.0 * (col_score + co_ref[0] - 0.05))
  cm_ref[...] = C * (rm[:, None] * cm[None, :])
  rm_ref[...] = rm[None, :]
  cmk_ref[...] = cm[None, :]


def kernel(edge_index1, x1, edge_index2, x2, W1, b1, W2, b2,
           row_offsets, col_offsets):
  src_all = jnp.concatenate([edge_index1[0], edge_index2[0]])
  dst_all = jnp.concatenate([edge_index1[1], edge_index2[1]])

  deg = _build_degree_kernel()(src_all, dst_all).reshape(4, NP)

  xs = jnp.stack([x1, x2])
  prep = pl.pallas_call(
      _prep_body,
      grid=(2,),
      in_specs=[
          pl.BlockSpec((1, N, D), lambda g: (g, 0, 0)),
          pl.BlockSpec((D, D), lambda g: (0, 0)),
          pl.BlockSpec((4, NP), lambda g: (0, 0)),
      ],
      out_specs=pl.BlockSpec((1, N, D), lambda g: (g, 0, 0)),
      out_shape=jax.ShapeDtypeStruct((2, N, D), jnp.float32),
  )
  hs = prep(xs, W1, deg)

  conv = _build_conv_kernel()
  agg1 = conv(hs.reshape(2 * N, D), src_all, dst_all)

  mid = pl.pallas_call(
      _mid_body,
      grid=(2,),
      in_specs=[
          pl.BlockSpec((1, NP, D), lambda g: (g, 0, 0)),
          pl.BlockSpec((4, NP), lambda g: (0, 0)),
          pl.BlockSpec((D, D), lambda g: (0, 0)),
          pl.BlockSpec((1, D), lambda g: (0, 0)),
      ],
      out_specs=pl.BlockSpec((1, N, D), lambda g: (g, 0, 0)),
      out_shape=jax.ShapeDtypeStruct((2, N, D), jnp.float32),
  )
  hsb = mid(agg1.reshape(2, NP, D), deg, W2, b1.reshape(1, D))

  agg2 = conv(hsb.reshape(2 * N, D), src_all, dst_all)

  tail = pl.pallas_call(
      _tail_body,
      out_shape=(
          jax.ShapeDtypeStruct((D, D), jnp.float32),
          jax.ShapeDtypeStruct((1, D), jnp.float32),
          jax.ShapeDtypeStruct((1, D), jnp.float32),
          jax.ShapeDtypeStruct((N, D), jnp.float32),
          jax.ShapeDtypeStruct((N, D), jnp.float32),
      ),
  )
  C_masked, rm, cm, z1, z2 = tail(
      agg2.reshape(2, NP, D), deg, b2.reshape(1, D),
      row_offsets.reshape(1, D), col_offsets.reshape(1, D))

  return (C_masked, rm.reshape(D), cm.reshape(D), z1, z2)


# speedup vs baseline: 1.0785x; 1.0785x over previous
import functools

import jax
import jax.numpy as jnp
from jax import lax
from jax.experimental import pallas as pl
from jax.experimental.pallas import tpu as pltpu
from jax.experimental.pallas import tpu_sc as plsc

N = 10000
E = 320000
D = 128

NC = 2
NS = 16
L = 16

NP = 10240
W_RED = NP // NS

EPT = E // NS
CHUNK = 128
NFULL = EPT // CHUNK
REM = EPT - NFULL * CHUNK

GROUP = 26

DEG_CH = 800

_SC_PARAMS = pltpu.CompilerParams(needs_layout_passes=False)


def _build_degree_kernel():
  mesh = plsc.VectorSubcoreMesh(core_axis_name="c", subcore_axis_name="s")

  @functools.partial(
      pl.kernel,
      out_type=jax.ShapeDtypeStruct((4 * NP,), jnp.float32),
      mesh=mesh,
      compiler_params=_SC_PARAMS,
      scratch_types=[
          pltpu.VMEM((DEG_CH,), jnp.int32),
          pltpu.VMEM((DEG_CH,), jnp.int32),
          pltpu.VMEM((NP,), jnp.float32),
          pltpu.VMEM((NP,), jnp.float32),
          pltpu.VMEM_SHARED((NS * NP,), jnp.float32),
          pltpu.VMEM_SHARED((NS * NP,), jnp.float32),
          pltpu.VMEM((NS * W_RED,), jnp.float32),
          pltpu.VMEM((W_RED,), jnp.float32),
      ],
  )
  def deg_kernel(src_hbm, dst_hbm, out_hbm, src_v, dst_v, ho, hi,
                 parts_o, parts_i, redbuf, res):
    c = lax.axis_index("c")
    s = lax.axis_index("s")
    zeros16 = jnp.zeros((L,), jnp.float32)

    @pl.loop(0, NP // L)
    def _zero(i):
      ho[pl.ds(i * L, L)] = zeros16
      hi[pl.ds(i * L, L)] = zeros16

    base = c * E + s * EPT

    @pl.loop(0, EPT // DEG_CH)
    def _edges(i):
      pltpu.sync_copy(src_hbm.at[pl.ds(base + i * DEG_CH, DEG_CH)], src_v)
      pltpu.sync_copy(dst_hbm.at[pl.ds(base + i * DEG_CH, DEG_CH)], dst_v)

      @pl.loop(0, DEG_CH // L, unroll=4)
      def _vecs(j):
        sidx = src_v[pl.ds(j * L, L)]
        scnt, slast = plsc.scan_count(sidx)
        plsc.addupdate_scatter(ho, [sidx], scnt.astype(jnp.float32),
                               mask=slast)
        didx = dst_v[pl.ds(j * L, L)]
        dcnt, dlast = plsc.scan_count(didx)
        plsc.addupdate_scatter(hi, [didx], dcnt.astype(jnp.float32),
                               mask=dlast)

    pltpu.sync_copy(ho, parts_o.at[pl.ds(s * NP, NP)])
    pltpu.sync_copy(hi, parts_i.at[pl.ds(s * NP, NP)])
    plsc.subcore_barrier()

    for h, parts in ((0, parts_o), (1, parts_i)):
      for p in range(NS):
        pltpu.sync_copy(parts.at[pl.ds(p * NP + s * W_RED, W_RED)],
                        redbuf.at[pl.ds(p * W_RED, W_RED)])

      @pl.loop(0, W_RED // L)
      def _red(j):
        acc = redbuf[pl.ds(j * L, L)]
        for p in range(1, NS):
          acc = acc + redbuf[pl.ds(p * W_RED + j * L, L)]
        res[pl.ds(j * L, L)] = acc

      pltpu.sync_copy(res,
                      out_hbm.at[pl.ds((2 * c + h) * NP + s * W_RED, W_RED)])

  return deg_kernel


def _build_conv_kernel():
  mesh = plsc.VectorSubcoreMesh(core_axis_name="c", subcore_axis_name="s")

  @functools.partial(
      pl.kernel,
      out_type=jax.ShapeDtypeStruct((2 * NP, D), jnp.float32),
      mesh=mesh,
      compiler_params=_SC_PARAMS,
      scratch_types=[
          pltpu.VMEM((GROUP * CHUNK,), jnp.int32),
          pltpu.VMEM((GROUP * CHUNK,), jnp.int32),
          pltpu.VMEM((CHUNK,), jnp.int32),
          pltpu.VMEM((CHUNK,), jnp.int32),
          pltpu.VMEM((CHUNK, D), jnp.float32),
          pltpu.VMEM((CHUNK, D), jnp.float32),
          pltpu.VMEM((REM,), jnp.int32),
          pltpu.VMEM((REM,), jnp.int32),
          pltpu.VMEM((REM, D), jnp.float32),
          pltpu.VMEM_SHARED((NP, D), jnp.float32),
          pltpu.SemaphoreType.DMA,
          pltpu.SemaphoreType.DMA,
          pltpu.SemaphoreType.DMA,
          pltpu.SemaphoreType.DMA,
          pltpu.SemaphoreType.DMA,
      ],
  )
  def conv_kernel(hs_hbm, src_hbm, dst_hbm, out_hbm, sv_g, dv_g, dv0, dv1,
                  rows0, rows1, src_r, dst_r, rows_r, acc,
                  gsem0, gsem1, ssem0, ssem1, rsem):
    c = lax.axis_index("c")
    s = lax.axis_index("s")
    zeros16 = jnp.zeros((L,), jnp.float32)

    ebase = c * E + s * EPT
    roff = c * N

    @pl.loop(0, CHUNK)
    def _zfill(r):
      for jj in range(D // L):
        rows0[r, pl.ds(jj * L, L)] = zeros16

    row0 = s * W_RED
    for k in range(W_RED // CHUNK):
      pltpu.sync_copy(rows0, acc.at[pl.ds(row0 + k * CHUNK, CHUNK)])
    plsc.subcore_barrier()

    slots = ((dv0, rows0, gsem0, ssem0), (dv1, rows1, gsem1, ssem1))

    def do_chunk(q, slot, drain):
      dv, rows, gsem, ssem = slots[slot]
      if drain:
        pltpu.make_async_copy(rows, acc.at[dv], ssem).wait()
      for j in range(CHUNK // L):
        dv[pl.ds(j * L, L)] = dv_g[pl.ds(q * CHUNK + j * L, L)]
      pltpu.async_copy(hs_hbm.at[sv_g.at[pl.ds(q * CHUNK, CHUNK)]],
                       rows, gsem).wait()
      pltpu.async_copy(rows, acc.at[dv], ssem, add=True)

    def load_group(g):
      b0 = ebase + g * GROUP * CHUNK
      pltpu.sync_copy(src_hbm.at[pl.ds(b0, GROUP * CHUNK)], sv_g)
      pltpu.sync_copy(dst_hbm.at[pl.ds(b0, GROUP * CHUNK)], dv_g)

      @pl.loop(0, GROUP * CHUNK // L)
      def _adj(i):
        sv_g[pl.ds(i * L, L)] = sv_g[pl.ds(i * L, L)] + roff

    load_group(0)
    do_chunk(0, 0, False)
    do_chunk(1, 1, False)
    for q in range(2, GROUP):
      do_chunk(q, q % 2, True)

    @pl.loop(1, NFULL // GROUP)
    def _groups(g):
      load_group(g)
      for q in range(GROUP):
        do_chunk(q, q % 2, True)

    b0 = ebase + NFULL * CHUNK
    pltpu.sync_copy(src_hbm.at[pl.ds(b0, REM)], src_r)
    pltpu.sync_copy(dst_hbm.at[pl.ds(b0, REM)], dst_r)
    for j in range(REM // L):
      src_r[pl.ds(j * L, L)] = src_r[pl.ds(j * L, L)] + roff
    pltpu.async_copy(hs_hbm.at[src_r], rows_r, rsem).wait()
    pltpu.sync_copy(rows_r, acc.at[dst_r], add=True)

    for dv, rows, gsem, ssem in slots:
      pltpu.make_async_copy(rows, acc.at[dv], ssem).wait()

    plsc.subcore_barrier()
    pltpu.sync_copy(acc.at[pl.ds(row0, W_RED)],
                    out_hbm.at[pl.ds(c * NP + row0, W_RED)])

  return conv_kernel


def _ns_nd(deg_full):
  g = pl.program_id(0)
  rs = lax.rsqrt(jnp.maximum(deg_full[:, :N], 1.0))
  ns = jnp.where(g == 0, rs[0], rs[2])
  nd = jnp.where(g == 0, rs[1], rs[3])
  return ns, nd


def _prep_body(xs_ref, w1_ref, deg_ref, hs_ref):
  ns, _ = _ns_nd(deg_ref[...])
  h = jnp.dot(xs_ref[0], w1_ref[...], preferred_element_type=jnp.float32, precision=lax.Precision.HIGHEST)
  hs_ref[0] = h * ns[:, None]


def _mid_body(agg_ref, deg_ref, w2_ref, b1_ref, out_ref):
  ns, nd = _ns_nd(deg_ref[...])
  a = agg_ref[0, :N]
  h = jnp.maximum(a * nd[:, None] + b1_ref[...], 0.0)
  h2 = jnp.dot(h, w2_ref[...], preferred_element_type=jnp.float32, precision=lax.Precision.HIGHEST)
  out_ref[0] = h2 * ns[:, None]


def _sigmoid(x):
  return 1.0 / (1.0 + jnp.exp(-x))


def _tail_body(agg_ref, deg_ref, b2_ref, ro_ref, co_ref,
               cm_ref, rm_ref, cmk_ref, z1_ref, z2_ref):
  degs = deg_ref[...]
  nd1 = lax.rsqrt(jnp.maximum(degs[1, :N], 1.0))
  nd2 = lax.rsqrt(jnp.maximum(degs[3, :N], 1.0))
  b2 = b2_ref[...]
  z1 = agg_ref[0, :N] * nd1[:, None] + b2
  z2 = agg_ref[1, :N] * nd2[:, None] + b2
  z1_ref[...] = z1
  z2_ref[...] = z2
  n = jnp.float32(N)
  mu1 = jnp.sum(z1, axis=0) / n
  mu2 = jnp.sum(z2, axis=0) / n
  s1 = jnp.sum(z1 * z1, axis=0)
  s2 = jnp.sum(z2 * z2, axis=0)
  var1 = (s1 - n * mu1 * mu1) / (n - 1.0)
  var2 = (s2 - n * mu2 * mu2) / (n - 1.0)
  sd1 = jnp.sqrt(jnp.maximum(var1, 0.0)) + 1e-6
  sd2 = jnp.sqrt(jnp.maximum(var2, 0.0)) + 1e-6
  S = lax.dot_general(z1, z2, (((0,), (0,)), ((), ())),
                      preferred_element_type=jnp.float32,
                      precision=lax.Precision.HIGHEST)
  C = (S - n * mu1[:, None] * mu2[None, :]) / (n * sd1[:, None] * sd2[None, :])
  row_score = jnp.mean(jnp.abs(C), axis=1)
  col_score = jnp.mean(jnp.abs(C), axis=0)
  rm = _sigmoid(50.0 * (row_score + ro_ref[0] - 0.05))
  cm = _sigmoid(50.0 * (col_score + co_ref[0] - 0.05))
  cm_ref[...] = C * (rm[:, None] * cm[None, :])
  rm_ref[...] = rm[None, :]
  cmk_ref[...] = cm[None, :]


def kernel(edge_index1, x1, edge_index2, x2, W1, b1, W2, b2,
           row_offsets, col_offsets):
  src_all = jnp.concatenate([edge_index1[0], edge_index2[0]])
  dst_all = jnp.concatenate([edge_index1[1], edge_index2[1]])

  deg = _build_degree_kernel()(src_all, dst_all).reshape(4, NP)

  xs = jnp.stack([x1, x2])
  prep = pl.pallas_call(
      _prep_body,
      grid=(2,),
      in_specs=[
          pl.BlockSpec((1, N, D), lambda g: (g, 0, 0)),
          pl.BlockSpec((D, D), lambda g: (0, 0)),
          pl.BlockSpec((4, NP), lambda g: (0, 0)),
      ],
      out_specs=pl.BlockSpec((1, N, D), lambda g: (g, 0, 0)),
      out_shape=jax.ShapeDtypeStruct((2, N, D), jnp.float32),
  )
  hs = prep(xs, W1, deg)

  conv = _build_conv_kernel()
  agg1 = conv(hs.reshape(2 * N, D), src_all, dst_all)

  mid = pl.pallas_call(
      _mid_body,
      grid=(2,),
      in_specs=[
          pl.BlockSpec((1, NP, D), lambda g: (g, 0, 0)),
          pl.BlockSpec((4, NP), lambda g: (0, 0)),
          pl.BlockSpec((D, D), lambda g: (0, 0)),
          pl.BlockSpec((1, D), lambda g: (0, 0)),
      ],
      out_specs=pl.BlockSpec((1, N, D), lambda g: (g, 0, 0)),
      out_shape=jax.ShapeDtypeStruct((2, N, D), jnp.float32),
  )
  hsb = mid(agg1.reshape(2, NP, D), deg, W2, b1.reshape(1, D))

  agg2 = conv(hsb.reshape(2 * N, D), src_all, dst_all)

  tail = pl.pallas_call(
      _tail_body,
      out_shape=(
          jax.ShapeDtypeStruct((D, D), jnp.float32),
          jax.ShapeDtypeStruct((1, D), jnp.float32),
          jax.ShapeDtypeStruct((1, D), jnp.float32),
          jax.ShapeDtypeStruct((N, D), jnp.float32),
          jax.ShapeDtypeStruct((N, D), jnp.float32),
      ),
  )
  C_masked, rm, cm, z1, z2 = tail(
      agg2.reshape(2, NP, D), deg, b2.reshape(1, D),
      row_offsets.reshape(1, D), col_offsets.reshape(1, D))

  return (C_masked, rm.reshape(D), cm.reshape(D), z1, z2)


# speedup vs baseline: 1.0845x; 1.0056x over previous
import functools

import jax
import jax.numpy as jnp
from jax import lax
from jax.experimental import pallas as pl
from jax.experimental.pallas import tpu as pltpu
from jax.experimental.pallas import tpu_sc as plsc

N = 10000
E = 320000
D = 128

NC = 2
NS = 16
L = 16

NP = 10240
W_RED = NP // NS

EPT = E // NS
CHUNK = 128
NFULL = EPT // CHUNK
REM = EPT - NFULL * CHUNK

GROUP = 26

DEG_CH = 800

_SC_PARAMS = pltpu.CompilerParams(needs_layout_passes=False)


def _build_degree_kernel():
  mesh = plsc.VectorSubcoreMesh(core_axis_name="c", subcore_axis_name="s")

  @functools.partial(
      pl.kernel,
      out_type=jax.ShapeDtypeStruct((4 * NP,), jnp.float32),
      mesh=mesh,
      compiler_params=_SC_PARAMS,
      scratch_types=[
          pltpu.VMEM((DEG_CH,), jnp.int32),
          pltpu.VMEM((DEG_CH,), jnp.int32),
          pltpu.VMEM((NP,), jnp.float32),
          pltpu.VMEM((NP,), jnp.float32),
          pltpu.VMEM_SHARED((NS * NP,), jnp.float32),
          pltpu.VMEM_SHARED((NS * NP,), jnp.float32),
          pltpu.VMEM((NS * W_RED,), jnp.float32),
          pltpu.VMEM((W_RED,), jnp.float32),
      ],
  )
  def deg_kernel(src_hbm, dst_hbm, out_hbm, src_v, dst_v, ho, hi,
                 parts_o, parts_i, redbuf, res):
    c = lax.axis_index("c")
    s = lax.axis_index("s")
    zeros16 = jnp.zeros((L,), jnp.float32)

    @pl.loop(0, NP // L)
    def _zero(i):
      ho[pl.ds(i * L, L)] = zeros16
      hi[pl.ds(i * L, L)] = zeros16

    base = c * E + s * EPT

    @pl.loop(0, EPT // DEG_CH)
    def _edges(i):
      pltpu.sync_copy(src_hbm.at[pl.ds(base + i * DEG_CH, DEG_CH)], src_v)
      pltpu.sync_copy(dst_hbm.at[pl.ds(base + i * DEG_CH, DEG_CH)], dst_v)

      @pl.loop(0, DEG_CH // L, unroll=4)
      def _vecs(j):
        sidx = src_v[pl.ds(j * L, L)]
        scnt, slast = plsc.scan_count(sidx)
        plsc.addupdate_scatter(ho, [sidx], scnt.astype(jnp.float32),
                               mask=slast)
        didx = dst_v[pl.ds(j * L, L)]
        dcnt, dlast = plsc.scan_count(didx)
        plsc.addupdate_scatter(hi, [didx], dcnt.astype(jnp.float32),
                               mask=dlast)

    pltpu.sync_copy(ho, parts_o.at[pl.ds(s * NP, NP)])
    pltpu.sync_copy(hi, parts_i.at[pl.ds(s * NP, NP)])
    plsc.subcore_barrier()

    for h, parts in ((0, parts_o), (1, parts_i)):
      for p in range(NS):
        pltpu.sync_copy(parts.at[pl.ds(p * NP + s * W_RED, W_RED)],
                        redbuf.at[pl.ds(p * W_RED, W_RED)])

      @pl.loop(0, W_RED // L)
      def _red(j):
        acc = redbuf[pl.ds(j * L, L)]
        for p in range(1, NS):
          acc = acc + redbuf[pl.ds(p * W_RED + j * L, L)]
        res[pl.ds(j * L, L)] = acc

      pltpu.sync_copy(res,
                      out_hbm.at[pl.ds((2 * c + h) * NP + s * W_RED, W_RED)])

  return deg_kernel


def _build_conv_kernel():
  mesh = plsc.VectorSubcoreMesh(core_axis_name="c", subcore_axis_name="s")

  @functools.partial(
      pl.kernel,
      out_type=jax.ShapeDtypeStruct((2 * NP, D), jnp.float32),
      mesh=mesh,
      compiler_params=_SC_PARAMS,
      scratch_types=[
          pltpu.VMEM((GROUP * CHUNK,), jnp.int32),
          pltpu.VMEM((GROUP * CHUNK,), jnp.int32),
          pltpu.VMEM((CHUNK,), jnp.int32),
          pltpu.VMEM((CHUNK,), jnp.int32),
          pltpu.VMEM((CHUNK, D), jnp.float32),
          pltpu.VMEM((CHUNK, D), jnp.float32),
          pltpu.VMEM((REM,), jnp.int32),
          pltpu.VMEM((REM,), jnp.int32),
          pltpu.VMEM((REM, D), jnp.float32),
          pltpu.VMEM_SHARED((NP, D), jnp.float32),
          pltpu.SemaphoreType.DMA,
          pltpu.SemaphoreType.DMA,
          pltpu.SemaphoreType.DMA,
          pltpu.SemaphoreType.DMA,
          pltpu.SemaphoreType.DMA,
      ],
  )
  def conv_kernel(hs_hbm, src_hbm, dst_hbm, out_hbm, sv_g, dv_g, dv0, dv1,
                  rows0, rows1, src_r, dst_r, rows_r, acc,
                  gsem0, gsem1, ssem0, ssem1, rsem):
    c = lax.axis_index("c")
    s = lax.axis_index("s")
    zeros16 = jnp.zeros((L,), jnp.float32)

    ebase = c * E + s * EPT
    roff = c * N

    @pl.loop(0, CHUNK)
    def _zfill(r):
      for jj in range(D // L):
        rows0[r, pl.ds(jj * L, L)] = zeros16

    row0 = s * W_RED
    for k in range(W_RED // CHUNK):
      pltpu.sync_copy(rows0, acc.at[pl.ds(row0 + k * CHUNK, CHUNK)])
    plsc.subcore_barrier()

    slots = ((dv0, rows0, gsem0, ssem0), (dv1, rows1, gsem1, ssem1))

    def do_chunk(q, slot, drain):
      dv, rows, gsem, ssem = slots[slot]
      if drain:
        pltpu.make_async_copy(rows, acc.at[dv], ssem).wait()
      for j in range(CHUNK // L):
        dv[pl.ds(j * L, L)] = dv_g[pl.ds(q * CHUNK + j * L, L)]
      pltpu.async_copy(hs_hbm.at[sv_g.at[pl.ds(q * CHUNK, CHUNK)]],
                       rows, gsem).wait()
      pltpu.async_copy(rows, acc.at[dv], ssem, add=True)

    def load_group(g):
      b0 = ebase + g * GROUP * CHUNK
      pltpu.sync_copy(src_hbm.at[pl.ds(b0, GROUP * CHUNK)], sv_g)
      pltpu.sync_copy(dst_hbm.at[pl.ds(b0, GROUP * CHUNK)], dv_g)

      @pl.loop(0, GROUP * CHUNK // L)
      def _adj(i):
        sv_g[pl.ds(i * L, L)] = sv_g[pl.ds(i * L, L)] + roff

    load_group(0)
    do_chunk(0, 0, False)
    do_chunk(1, 1, False)
    for q in range(2, GROUP):
      do_chunk(q, q % 2, True)

    @pl.loop(1, NFULL // GROUP)
    def _groups(g):
      load_group(g)
      for q in range(GROUP):
        do_chunk(q, q % 2, True)

    b0 = ebase + NFULL * CHUNK
    pltpu.sync_copy(src_hbm.at[pl.ds(b0, REM)], src_r)
    pltpu.sync_copy(dst_hbm.at[pl.ds(b0, REM)], dst_r)
    for j in range(REM // L):
      src_r[pl.ds(j * L, L)] = src_r[pl.ds(j * L, L)] + roff
    pltpu.async_copy(hs_hbm.at[src_r], rows_r, rsem).wait()
    pltpu.sync_copy(rows_r, acc.at[dst_r], add=True)

    for dv, rows, gsem, ssem in slots:
      pltpu.make_async_copy(rows, acc.at[dv], ssem).wait()

    plsc.subcore_barrier()
    pltpu.sync_copy(acc.at[pl.ds(row0, W_RED)],
                    out_hbm.at[pl.ds(c * NP + row0, W_RED)])

  return conv_kernel


def _ns_nd(deg_full):
  g = pl.program_id(0)
  rs = lax.rsqrt(jnp.maximum(deg_full[:, :N], 1.0))
  ns = jnp.where(g == 0, rs[0], rs[2])
  nd = jnp.where(g == 0, rs[1], rs[3])
  return ns, nd


def _mm_body(xs_ref, w1_ref, h_ref):
  h_ref[0] = jnp.dot(xs_ref[0], w1_ref[...],
                     preferred_element_type=jnp.float32,
                     precision=lax.Precision.HIGHEST)


def _scale_body(h_ref, deg_ref, hs_ref):
  ns, _ = _ns_nd(deg_ref[...])
  hs_ref[0] = h_ref[0] * ns[:, None]


def _mid_body(agg_ref, deg_ref, w2_ref, b1_ref, out_ref):
  ns, nd = _ns_nd(deg_ref[...])
  a = agg_ref[0, :N]
  h = jnp.maximum(a * nd[:, None] + b1_ref[...], 0.0)
  h2 = jnp.dot(h, w2_ref[...], preferred_element_type=jnp.float32, precision=lax.Precision.HIGHEST)
  out_ref[0] = h2 * ns[:, None]


def _sigmoid(x):
  return 1.0 / (1.0 + jnp.exp(-x))


def _tail_body(agg_ref, deg_ref, b2_ref, ro_ref, co_ref,
               cm_ref, rm_ref, cmk_ref, z1_ref, z2_ref):
  degs = deg_ref[...]
  nd1 = lax.rsqrt(jnp.maximum(degs[1, :N], 1.0))
  nd2 = lax.rsqrt(jnp.maximum(degs[3, :N], 1.0))
  b2 = b2_ref[...]
  z1 = agg_ref[0, :N] * nd1[:, None] + b2
  z2 = agg_ref[1, :N] * nd2[:, None] + b2
  z1_ref[...] = z1
  z2_ref[...] = z2
  n = jnp.float32(N)
  mu1 = jnp.sum(z1, axis=0) / n
  mu2 = jnp.sum(z2, axis=0) / n
  s1 = jnp.sum(z1 * z1, axis=0)
  s2 = jnp.sum(z2 * z2, axis=0)
  var1 = (s1 - n * mu1 * mu1) / (n - 1.0)
  var2 = (s2 - n * mu2 * mu2) / (n - 1.0)
  sd1 = jnp.sqrt(jnp.maximum(var1, 0.0)) + 1e-6
  sd2 = jnp.sqrt(jnp.maximum(var2, 0.0)) + 1e-6
  S = lax.dot_general(z1, z2, (((0,), (0,)), ((), ())),
                      preferred_element_type=jnp.float32,
                      precision=lax.Precision.HIGHEST)
  C = (S - n * mu1[:, None] * mu2[None, :]) / (n * sd1[:, None] * sd2[None, :])
  row_score = jnp.mean(jnp.abs(C), axis=1)
  col_score = jnp.mean(jnp.abs(C), axis=0)
  rm = _sigmoid(50.0 * (row_score + ro_ref[0] - 0.05))
  cm = _sigmoid(50.0 * (col_score + co_ref[0] - 0.05))
  cm_ref[...] = C * (rm[:, None] * cm[None, :])
  rm_ref[...] = rm[None, :]
  cmk_ref[...] = cm[None, :]


def kernel(edge_index1, x1, edge_index2, x2, W1, b1, W2, b2,
           row_offsets, col_offsets):
  src_all = jnp.concatenate([edge_index1[0], edge_index2[0]])
  dst_all = jnp.concatenate([edge_index1[1], edge_index2[1]])

  deg = _build_degree_kernel()(src_all, dst_all).reshape(4, NP)

  xs = jnp.stack([x1, x2])
  mm = pl.pallas_call(
      _mm_body,
      grid=(2,),
      in_specs=[
          pl.BlockSpec((1, N, D), lambda g: (g, 0, 0)),
          pl.BlockSpec((D, D), lambda g: (0, 0)),
      ],
      out_specs=pl.BlockSpec((1, N, D), lambda g: (g, 0, 0)),
      out_shape=jax.ShapeDtypeStruct((2, N, D), jnp.float32),
  )
  h1 = mm(xs, W1)
  scale = pl.pallas_call(
      _scale_body,
      grid=(2,),
      in_specs=[
          pl.BlockSpec((1, N, D), lambda g: (g, 0, 0)),
          pl.BlockSpec((4, NP), lambda g: (0, 0)),
      ],
      out_specs=pl.BlockSpec((1, N, D), lambda g: (g, 0, 0)),
      out_shape=jax.ShapeDtypeStruct((2, N, D), jnp.float32),
  )
  hs = scale(h1, deg)

  conv = _build_conv_kernel()
  agg1 = conv(hs.reshape(2 * N, D), src_all, dst_all)

  mid = pl.pallas_call(
      _mid_body,
      grid=(2,),
      in_specs=[
          pl.BlockSpec((1, NP, D), lambda g: (g, 0, 0)),
          pl.BlockSpec((4, NP), lambda g: (0, 0)),
          pl.BlockSpec((D, D), lambda g: (0, 0)),
          pl.BlockSpec((1, D), lambda g: (0, 0)),
      ],
      out_specs=pl.BlockSpec((1, N, D), lambda g: (g, 0, 0)),
      out_shape=jax.ShapeDtypeStruct((2, N, D), jnp.float32),
  )
  hsb = mid(agg1.reshape(2, NP, D), deg, W2, b1.reshape(1, D))

  agg2 = conv(hsb.reshape(2 * N, D), src_all, dst_all)

  tail = pl.pallas_call(
      _tail_body,
      out_shape=(
          jax.ShapeDtypeStruct((D, D), jnp.float32),
          jax.ShapeDtypeStruct((1, D), jnp.float32),
          jax.ShapeDtypeStruct((1, D), jnp.float32),
          jax.ShapeDtypeStruct((N, D), jnp.float32),
          jax.ShapeDtypeStruct((N, D), jnp.float32),
      ),
  )
  C_masked, rm, cm, z1, z2 = tail(
      agg2.reshape(2, NP, D), deg, b2.reshape(1, D),
      row_offsets.reshape(1, D), col_offsets.reshape(1, D))

  return (C_masked, rm.reshape(D), cm.reshape(D), z1, z2)


# speedup vs baseline: 1.1283x; 1.0404x over previous
import functools

import jax
import jax.numpy as jnp
from jax import lax
from jax.experimental import pallas as pl
from jax.experimental.pallas import tpu as pltpu
from jax.experimental.pallas import tpu_sc as plsc

N = 10000
E = 320000
D = 128

NC = 2
NS = 16
L = 16

NP = 10240
W_RED = NP // NS

EPT = E // NS
CHUNK = 128
NFULL = EPT // CHUNK
REM = EPT - NFULL * CHUNK

GROUP = 26

DEG_CH = 2000

_SC_PARAMS = pltpu.CompilerParams(needs_layout_passes=False)


def _build_degree_kernel():
  mesh = plsc.VectorSubcoreMesh(core_axis_name="c", subcore_axis_name="s")

  @functools.partial(
      pl.kernel,
      out_type=jax.ShapeDtypeStruct((4 * NP,), jnp.float32),
      mesh=mesh,
      compiler_params=_SC_PARAMS,
      scratch_types=[
          pltpu.VMEM((DEG_CH,), jnp.int32),
          pltpu.VMEM((DEG_CH,), jnp.int32),
          pltpu.VMEM((NP,), jnp.float32),
          pltpu.VMEM((NP,), jnp.float32),
          pltpu.VMEM_SHARED((NS * NP,), jnp.float32),
          pltpu.VMEM_SHARED((NS * NP,), jnp.float32),
          pltpu.VMEM((NS * W_RED,), jnp.float32),
          pltpu.VMEM((W_RED,), jnp.float32),
      ],
  )
  def deg_kernel(src_hbm, dst_hbm, out_hbm, src_v, dst_v, ho, hi,
                 parts_o, parts_i, redbuf, res):
    c = lax.axis_index("c")
    s = lax.axis_index("s")
    zeros16 = jnp.zeros((L,), jnp.float32)

    @pl.loop(0, NP // L)
    def _zero(i):
      ho[pl.ds(i * L, L)] = zeros16
      hi[pl.ds(i * L, L)] = zeros16

    base = c * E + s * EPT

    @pl.loop(0, EPT // DEG_CH)
    def _edges(i):
      pltpu.sync_copy(src_hbm.at[pl.ds(base + i * DEG_CH, DEG_CH)], src_v)
      pltpu.sync_copy(dst_hbm.at[pl.ds(base + i * DEG_CH, DEG_CH)], dst_v)

      @pl.loop(0, DEG_CH // L, unroll=4)
      def _vecs(j):
        sidx = src_v[pl.ds(j * L, L)]
        scnt, slast = plsc.scan_count(sidx)
        plsc.addupdate_scatter(ho, [sidx], scnt.astype(jnp.float32),
                               mask=slast)
        didx = dst_v[pl.ds(j * L, L)]
        dcnt, dlast = plsc.scan_count(didx)
        plsc.addupdate_scatter(hi, [didx], dcnt.astype(jnp.float32),
                               mask=dlast)

    pltpu.sync_copy(ho, parts_o.at[pl.ds(s * NP, NP)])
    pltpu.sync_copy(hi, parts_i.at[pl.ds(s * NP, NP)])
    plsc.subcore_barrier()

    for h, parts in ((0, parts_o), (1, parts_i)):
      for p in range(NS):
        pltpu.sync_copy(parts.at[pl.ds(p * NP + s * W_RED, W_RED)],
                        redbuf.at[pl.ds(p * W_RED, W_RED)])

      @pl.loop(0, W_RED // L)
      def _red(j):
        acc = redbuf[pl.ds(j * L, L)]
        for p in range(1, NS):
          acc = acc + redbuf[pl.ds(p * W_RED + j * L, L)]
        res[pl.ds(j * L, L)] = acc

      pltpu.sync_copy(res,
                      out_hbm.at[pl.ds((2 * c + h) * NP + s * W_RED, W_RED)])

  return deg_kernel


def _build_conv_kernel():
  mesh = plsc.VectorSubcoreMesh(core_axis_name="c", subcore_axis_name="s")

  @functools.partial(
      pl.kernel,
      out_type=jax.ShapeDtypeStruct((2 * NP, D), jnp.float32),
      mesh=mesh,
      compiler_params=_SC_PARAMS,
      scratch_types=[
          pltpu.VMEM((GROUP * CHUNK,), jnp.int32),
          pltpu.VMEM((GROUP * CHUNK,), jnp.int32),
          pltpu.VMEM((CHUNK,), jnp.int32),
          pltpu.VMEM((CHUNK,), jnp.int32),
          pltpu.VMEM((CHUNK, D), jnp.float32),
          pltpu.VMEM((CHUNK, D), jnp.float32),
          pltpu.VMEM((REM,), jnp.int32),
          pltpu.VMEM((REM,), jnp.int32),
          pltpu.VMEM((REM, D), jnp.float32),
          pltpu.VMEM_SHARED((NP, D), jnp.float32),
          pltpu.SemaphoreType.DMA,
          pltpu.SemaphoreType.DMA,
          pltpu.SemaphoreType.DMA,
          pltpu.SemaphoreType.DMA,
          pltpu.SemaphoreType.DMA,
      ],
  )
  def conv_kernel(hs_hbm, src_hbm, dst_hbm, out_hbm, sv_g, dv_g, dv0, dv1,
                  rows0, rows1, src_r, dst_r, rows_r, acc,
                  gsem0, gsem1, ssem0, ssem1, rsem):
    c = lax.axis_index("c")
    s = lax.axis_index("s")
    zeros16 = jnp.zeros((L,), jnp.float32)

    ebase = c * E + s * EPT

    @pl.loop(0, CHUNK)
    def _zfill(r):
      for jj in range(D // L):
        rows0[r, pl.ds(jj * L, L)] = zeros16

    row0 = s * W_RED
    for k in range(W_RED // CHUNK):
      pltpu.sync_copy(rows0, acc.at[pl.ds(row0 + k * CHUNK, CHUNK)])
    plsc.subcore_barrier()

    slots = ((dv0, rows0, gsem0, ssem0), (dv1, rows1, gsem1, ssem1))

    def do_chunk(q, slot, drain):
      dv, rows, gsem, ssem = slots[slot]
      if drain:
        pltpu.make_async_copy(rows, acc.at[dv], ssem).wait()
      for j in range(CHUNK // L):
        dv[pl.ds(j * L, L)] = dv_g[pl.ds(q * CHUNK + j * L, L)]
      pltpu.async_copy(hs_hbm.at[sv_g.at[pl.ds(q * CHUNK, CHUNK)]],
                       rows, gsem).wait()
      pltpu.async_copy(rows, acc.at[dv], ssem, add=True)

    def load_group(g):
      b0 = ebase + g * GROUP * CHUNK
      pltpu.sync_copy(src_hbm.at[pl.ds(b0, GROUP * CHUNK)], sv_g)
      pltpu.sync_copy(dst_hbm.at[pl.ds(b0, GROUP * CHUNK)], dv_g)

    load_group(0)
    do_chunk(0, 0, False)
    do_chunk(1, 1, False)
    for q in range(2, GROUP):
      do_chunk(q, q % 2, True)

    @pl.loop(1, NFULL // GROUP)
    def _groups(g):
      load_group(g)
      for q in range(GROUP):
        do_chunk(q, q % 2, True)

    b0 = ebase + NFULL * CHUNK
    pltpu.sync_copy(src_hbm.at[pl.ds(b0, REM)], src_r)
    pltpu.sync_copy(dst_hbm.at[pl.ds(b0, REM)], dst_r)
    pltpu.async_copy(hs_hbm.at[src_r], rows_r, rsem).wait()
    pltpu.sync_copy(rows_r, acc.at[dst_r], add=True)

    for dv, rows, gsem, ssem in slots:
      pltpu.make_async_copy(rows, acc.at[dv], ssem).wait()

    plsc.subcore_barrier()
    pltpu.sync_copy(acc.at[pl.ds(row0, W_RED)],
                    out_hbm.at[pl.ds(c * NP + row0, W_RED)])

  return conv_kernel


def _ns_nd(deg_full):
  g = pl.program_id(0)
  rs = lax.rsqrt(jnp.maximum(deg_full[:, :N], 1.0))
  ns = jnp.where(g == 0, rs[0], rs[2])
  nd = jnp.where(g == 0, rs[1], rs[3])
  return ns, nd


def _mm_body(xs_ref, w1_ref, h_ref):
  h_ref[0] = jnp.dot(xs_ref[0], w1_ref[...],
                     preferred_element_type=jnp.float32,
                     precision=lax.Precision.HIGHEST)


def _scale_body(h_ref, deg_ref, hs_ref):
  ns, _ = _ns_nd(deg_ref[...])
  hs_ref[0] = h_ref[0] * ns[:, None]


def _mid_body(agg_ref, deg_ref, w2_ref, b1_ref, out_ref):
  ns, nd = _ns_nd(deg_ref[...])
  a = agg_ref[0, :N]
  h = jnp.maximum(a * nd[:, None] + b1_ref[...], 0.0)
  h2 = jnp.dot(h, w2_ref[...], preferred_element_type=jnp.float32, precision=lax.Precision.HIGHEST)
  out_ref[0] = h2 * ns[:, None]


def _sigmoid(x):
  return 1.0 / (1.0 + jnp.exp(-x))


def _tail_body(agg_ref, deg_ref, b2_ref, ro_ref, co_ref,
               cm_ref, rm_ref, cmk_ref, z1_ref, z2_ref):
  degs = deg_ref[...]
  nd1 = lax.rsqrt(jnp.maximum(degs[1, :N], 1.0))
  nd2 = lax.rsqrt(jnp.maximum(degs[3, :N], 1.0))
  b2 = b2_ref[...]
  z1 = agg_ref[0, :N] * nd1[:, None] + b2
  z2 = agg_ref[1, :N] * nd2[:, None] + b2
  z1_ref[...] = z1
  z2_ref[...] = z2
  n = jnp.float32(N)
  mu1 = jnp.sum(z1, axis=0) / n
  mu2 = jnp.sum(z2, axis=0) / n
  s1 = jnp.sum(z1 * z1, axis=0)
  s2 = jnp.sum(z2 * z2, axis=0)
  var1 = (s1 - n * mu1 * mu1) / (n - 1.0)
  var2 = (s2 - n * mu2 * mu2) / (n - 1.0)
  sd1 = jnp.sqrt(jnp.maximum(var1, 0.0)) + 1e-6
  sd2 = jnp.sqrt(jnp.maximum(var2, 0.0)) + 1e-6
  S = lax.dot_general(z1, z2, (((0,), (0,)), ((), ())),
                      preferred_element_type=jnp.float32,
                      precision=lax.Precision.HIGHEST)
  C = (S - n * mu1[:, None] * mu2[None, :]) / (n * sd1[:, None] * sd2[None, :])
  row_score = jnp.mean(jnp.abs(C), axis=1)
  col_score = jnp.mean(jnp.abs(C), axis=0)
  rm = _sigmoid(50.0 * (row_score + ro_ref[0] - 0.05))
  cm = _sigmoid(50.0 * (col_score + co_ref[0] - 0.05))
  cm_ref[...] = C * (rm[:, None] * cm[None, :])
  rm_ref[...] = rm[None, :]
  cmk_ref[...] = cm[None, :]


def kernel(edge_index1, x1, edge_index2, x2, W1, b1, W2, b2,
           row_offsets, col_offsets):
  src_all = jnp.concatenate([edge_index1[0], edge_index2[0]])
  dst_all = jnp.concatenate([edge_index1[1], edge_index2[1]])
  src_off = jnp.concatenate([edge_index1[0], edge_index2[0] + N])

  deg = _build_degree_kernel()(src_all, dst_all).reshape(4, NP)

  xs = jnp.stack([x1, x2])
  mm = pl.pallas_call(
      _mm_body,
      grid=(2,),
      in_specs=[
          pl.BlockSpec((1, N, D), lambda g: (g, 0, 0)),
          pl.BlockSpec((D, D), lambda g: (0, 0)),
      ],
      out_specs=pl.BlockSpec((1, N, D), lambda g: (g, 0, 0)),
      out_shape=jax.ShapeDtypeStruct((2, N, D), jnp.float32),
  )
  h1 = mm(xs, W1)
  scale = pl.pallas_call(
      _scale_body,
      grid=(2,),
      in_specs=[
          pl.BlockSpec((1, N, D), lambda g: (g, 0, 0)),
          pl.BlockSpec((4, NP), lambda g: (0, 0)),
      ],
      out_specs=pl.BlockSpec((1, N, D), lambda g: (g, 0, 0)),
      out_shape=jax.ShapeDtypeStruct((2, N, D), jnp.float32),
  )
  hs = scale(h1, deg)

  conv = _build_conv_kernel()
  agg1 = conv(hs.reshape(2 * N, D), src_off, dst_all)

  mid = pl.pallas_call(
      _mid_body,
      grid=(2,),
      in_specs=[
          pl.BlockSpec((1, NP, D), lambda g: (g, 0, 0)),
          pl.BlockSpec((4, NP), lambda g: (0, 0)),
          pl.BlockSpec((D, D), lambda g: (0, 0)),
          pl.BlockSpec((1, D), lambda g: (0, 0)),
      ],
      out_specs=pl.BlockSpec((1, N, D), lambda g: (g, 0, 0)),
      out_shape=jax.ShapeDtypeStruct((2, N, D), jnp.float32),
  )
  hsb = mid(agg1.reshape(2, NP, D), deg, W2, b1.reshape(1, D))

  agg2 = conv(hsb.reshape(2 * N, D), src_off, dst_all)

  tail = pl.pallas_call(
      _tail_body,
      out_shape=(
          jax.ShapeDtypeStruct((D, D), jnp.float32),
          jax.ShapeDtypeStruct((1, D), jnp.float32),
          jax.ShapeDtypeStruct((1, D), jnp.float32),
          jax.ShapeDtypeStruct((N, D), jnp.float32),
          jax.ShapeDtypeStruct((N, D), jnp.float32),
      ),
  )
  C_masked, rm, cm, z1, z2 = tail(
      agg2.reshape(2, NP, D), deg, b2.reshape(1, D),
      row_offsets.reshape(1, D), col_offsets.reshape(1, D))

  return (C_masked, rm.reshape(D), cm.reshape(D), z1, z2)
